# baseline (speedup 1.0000x reference)
import jax
import jax.numpy as jnp
from jax import lax
from jax.experimental import pallas as pl
from jax.experimental.pallas import tpu as pltpu
from jax.experimental.pallas import tpu_sc as plsc

H = 256
W = 256
DIM = 32
B = 8
N = 524288

NC = 2
NS = 16
L = 16

CHUNK = 32768
NCHUNK = B * H * W // CHUNK
NPASS = NCHUNK // NC
DUMP = CHUNK
SROWS = CHUNK + 8

EPT = N // NS
HALF = EPT // 2
G = 128
FSZ = HALF + G + 8
TRASH = HALF + G
SENT = HALF

CPT = CHUNK // NS
OSB = 256
NOSB = CPT // OSB
ZR = 128

RNE = 8388608.0

_SC_PARAMS = pltpu.CompilerParams(
    use_tc_tiling_on_sc=False, needs_layout_passes=False)


K1_ROWS = 512


def _lin_body(off_ref, x_ref, y_ref, lin_ref):
    g = pl.program_id(0)
    xv = x_ref[...]
    yv = y_ref[...]
    xr = (xv * float(W) + RNE) - RNE
    yr = (yv * float(H) + RNE) - RNE
    xi = jnp.minimum(jnp.maximum(xr, 0.0), float(W - 1)).astype(jnp.int32)
    yi = jnp.minimum(jnp.maximum(yr, 0.0), float(H - 1)).astype(jnp.int32)
    r = lax.broadcasted_iota(jnp.int32, xv.shape, 0)
    c = lax.broadcasted_iota(jnp.int32, xv.shape, 1)
    j = (g * K1_ROWS + r) * xv.shape[1] + c
    b = jnp.zeros(xv.shape, jnp.int32)
    for k in range(B - 1):
        b = b + jnp.where(j >= off_ref[k], 1, 0).astype(jnp.int32)
    lin_ref[...] = b * (H * W) + yi * W + xi


def _compute_lin(events, offsets):
    ncols = 512
    nrows = N // ncols
    xs = events[:, 0].reshape(nrows, ncols)
    ys = events[:, 1].reshape(nrows, ncols)
    lin = pl.pallas_call(
        _lin_body,
        out_shape=jax.ShapeDtypeStruct((nrows, ncols), jnp.int32),
        grid=(nrows // K1_ROWS,),
        in_specs=[
            pl.BlockSpec(memory_space=pltpu.SMEM),
            pl.BlockSpec((K1_ROWS, ncols), lambda g: (g, 0)),
            pl.BlockSpec((K1_ROWS, ncols), lambda g: (g, 0)),
        ],
        out_specs=pl.BlockSpec((K1_ROWS, ncols), lambda g: (g, 0)),
    )(offsets, xs, ys)
    return lin.reshape(N)


def _sc_body(lin_hbm, feat_hbm, out_hbm,
             lin_half, filt_ids, id_buf, cell_buf, feat_buf, ones_buf,
             s_buf, c_buf, inv_buf, plane_buf, zsum, zcnt,
             sums_sh, cnts_sh, gsem, psem):
    cid = lax.axis_index("c")
    sid = lax.axis_index("s")
    iota = lax.iota(jnp.int32, L)

    def fill_ones(k, _):
        ones_buf[pl.ds(k * L, L)] = jnp.ones((L,), jnp.float32)
        return 0
    lax.fori_loop(0, G // L, fill_ones, 0)

    def fill_zsum(k, _):
        r = k // (DIM // L)
        col = (k % (DIM // L)) * L
        zsum[r, pl.ds(col, L)] = jnp.zeros((L,), jnp.float32)
        return 0
    lax.fori_loop(0, ZR * (DIM // L), fill_zsum, 0)

    def fill_zcnt(k, _):
        zcnt[pl.ds(k * L, L)] = jnp.zeros((L,), jnp.float32)
        return 0
    lax.fori_loop(0, ZR // L, fill_zcnt, 0)

    lin_half[pl.ds(SENT, L)] = jnp.full((L,), jnp.int32(0x7FFFFFF), jnp.int32)

    def zero_sub(q, _):
        cell0 = sid * CPT + q * ZR
        pltpu.sync_copy(zsum, sums_sh.at[pl.ds(cell0, ZR)])
        pltpu.sync_copy(zcnt, cnts_sh.at[pl.ds(cell0, ZR)])
        return 0
    lax.fori_loop(0, CPT // ZR, zero_sub, 0)
    plsc.subcore_barrier()

    def do_pass(p, _):
        chunk_id = cid * NPASS + p

        def do_half(h, _):
            base_j = sid * EPT + h * HALF
            pltpu.sync_copy(lin_hbm.at[pl.ds(base_j, HALF)],
                            lin_half.at[pl.ds(0, HALF)])

            def filt(i, c):
                lin = lin_half[pl.ds(i * L, L)]
                m = lax.shift_right_logical(lin, 15) == chunk_id
                s = plsc.cumsum(m.astype(jnp.int32))
                pos = jnp.where(m, c + s - 1, TRASH)
                plsc.store_scatter(filt_ids, [pos], i * L + iota)
                return c + s[L - 1]
            c = lax.fori_loop(0, HALF // L, filt, jnp.int32(0))

            def pad(k, _):
                filt_ids[pl.ds(c + k * L, L)] = jnp.full((L,), SENT, jnp.int32)
                return 0
            lax.fori_loop(0, G // L, pad, 0)

            n_g = lax.shift_right_logical(c + (G - 1), 7)

            def group(g, _):
                base = g * G

                def cp(k, _):
                    loc = filt_ids[pl.ds(base + k * L, L)]
                    lin = plsc.load_gather(lin_half, [loc])
                    m = lax.shift_right_logical(lin, 15) == chunk_id
                    cell = jnp.where(m, lin & (CHUNK - 1), jnp.int32(DUMP))
                    gid = jnp.minimum(base_j + loc, jnp.int32(N - 1))
                    id_buf[pl.ds(k * L, L)] = gid
                    cell_buf[pl.ds(k * L, L)] = cell
                    return 0
                lax.fori_loop(0, G // L, cp, 0)
                pltpu.async_copy(feat_hbm.at[id_buf], feat_buf, gsem).wait()
                pltpu.sync_copy(feat_buf, sums_sh.at[cell_buf], add=True)
                pltpu.sync_copy(ones_buf, cnts_sh.at[cell_buf], add=True)
                return 0
            lax.fori_loop(0, n_g, group, 0)
            return 0
        lax.fori_loop(0, 2, do_half, 0)
        plsc.subcore_barrier()

        b_idx = lax.shift_right_logical(chunk_id, 1)
        yh = chunk_id & 1

        def out_sub(sub, _):
            cell0 = sid * CPT + sub * OSB
            pltpu.sync_copy(sums_sh.at[pl.ds(cell0, OSB)], s_buf)
            pltpu.sync_copy(cnts_sh.at[pl.ds(cell0, OSB)], c_buf)

            def inv_k(k, _):
                cv = c_buf[pl.ds(k * L, L)]
                inv_buf[pl.ds(k * L, L)] = 1.0 / jnp.maximum(cv, 1.0)
                return 0
            lax.fori_loop(0, OSB // L, inv_k, 0)

            out0 = (b_idx * DIM * H * W + yh * CHUNK
                    + sid * CPT + sub * OSB)

            def per_d(d, _):
                def tr(k, _):
                    rows = k * L + iota
                    v = plsc.load_gather(s_buf, [rows, jnp.full((L,), d, jnp.int32)])
                    v = v * inv_buf[pl.ds(k * L, L)]
                    plane_buf[d, pl.ds(k * L, L)] = v
                    return 0
                lax.fori_loop(0, OSB // L, tr, 0)
                off = out0 + d * (H * W)
                pltpu.async_copy(plane_buf.at[d], out_hbm.at[pl.ds(off, OSB)], psem)
                return 0
            lax.fori_loop(0, DIM, per_d, 0)

            def drain(d, _):
                off = out0 + d * (H * W)
                pltpu.make_async_copy(plane_buf.at[d], out_hbm.at[pl.ds(off, OSB)], psem).wait()
                return 0
            lax.fori_loop(0, DIM, drain, 0)

            def rezero(q, _):
                pltpu.sync_copy(zsum, sums_sh.at[pl.ds(cell0 + q * ZR, ZR)])
                pltpu.sync_copy(zcnt, cnts_sh.at[pl.ds(cell0 + q * ZR, ZR)])
                return 0
            lax.fori_loop(0, OSB // ZR, rezero, 0)
            return 0
        lax.fori_loop(0, NOSB, out_sub, 0)
        plsc.subcore_barrier()
        return 0
    lax.fori_loop(0, NPASS, do_pass, 0)


def kernel(events, features, offsets):
    lin = _compute_lin(events, offsets)
    mesh = plsc.VectorSubcoreMesh(core_axis_name="c", subcore_axis_name="s",
                                  num_cores=NC, num_subcores=NS)
    run = pl.kernel(
        _sc_body,
        out_type=jax.ShapeDtypeStruct((B * DIM * H * W,), jnp.float32),
        mesh=mesh,
        scratch_types=[
            pltpu.VMEM((HALF + L,), jnp.int32),
            pltpu.VMEM((FSZ,), jnp.int32),
            pltpu.VMEM((G,), jnp.int32),
            pltpu.VMEM((G,), jnp.int32),
            pltpu.VMEM((G, DIM), jnp.float32),
            pltpu.VMEM((G,), jnp.float32),
            pltpu.VMEM((OSB, DIM), jnp.float32),
            pltpu.VMEM((OSB,), jnp.float32),
            pltpu.VMEM((OSB,), jnp.float32),
            pltpu.VMEM((DIM, OSB), jnp.float32),
            pltpu.VMEM((ZR, DIM), jnp.float32),
            pltpu.VMEM((ZR,), jnp.float32),
            pltpu.VMEM_SHARED((SROWS, DIM), jnp.float32),
            pltpu.VMEM_SHARED((SROWS,), jnp.float32),
            pltpu.SemaphoreType.DMA,
            pltpu.SemaphoreType.DMA,
        ],
        compiler_params=_SC_PARAMS,
    )
    out = run(lin, features)
    return out.reshape(B, DIM, H, W)


# speedup vs baseline: 1.8865x; 1.8865x over previous
import jax
import jax.numpy as jnp
from jax import lax
from jax.experimental import pallas as pl
from jax.experimental.pallas import tpu as pltpu
from jax.experimental.pallas import tpu_sc as plsc

H = 256
W = 256
DIM = 32
B = 8
N = 524288

NC = 2
NS = 16
L = 16

CHUNK = 32768
NCHUNK = B * H * W // CHUNK
NPASS = NCHUNK // NC
DUMP = CHUNK
SROWS = CHUNK + 8

SEG = 4096
SEGSH = 12
G = 128
FSZ = SEG + G + 8
TRASH = SEG + G
SENT = SEG

CPT = CHUNK // NS
OSB = 256
NOSB = CPT // OSB
ZR = 128

RNE = 8388608.0

_SC_PARAMS = pltpu.CompilerParams(
    use_tc_tiling_on_sc=False, needs_layout_passes=False)


K1_ROWS = 512


def _lin_body(off_ref, x_ref, y_ref, lin_ref):
    g = pl.program_id(0)
    xv = x_ref[...]
    yv = y_ref[...]
    xr = (xv * float(W) + RNE) - RNE
    yr = (yv * float(H) + RNE) - RNE
    xi = jnp.minimum(jnp.maximum(xr, 0.0), float(W - 1)).astype(jnp.int32)
    yi = jnp.minimum(jnp.maximum(yr, 0.0), float(H - 1)).astype(jnp.int32)
    r = lax.broadcasted_iota(jnp.int32, xv.shape, 0)
    c = lax.broadcasted_iota(jnp.int32, xv.shape, 1)
    j = (g * K1_ROWS + r) * xv.shape[1] + c
    b = jnp.zeros(xv.shape, jnp.int32)
    for k in range(B - 1):
        b = b + jnp.where(j >= off_ref[k], 1, 0).astype(jnp.int32)
    lin_ref[...] = b * (H * W) + yi * W + xi


def _compute_lin(events, offsets):
    ncols = 512
    nrows = N // ncols
    xs = events[:, 0].reshape(nrows, ncols)
    ys = events[:, 1].reshape(nrows, ncols)
    lin = pl.pallas_call(
        _lin_body,
        out_shape=jax.ShapeDtypeStruct((nrows, ncols), jnp.int32),
        grid=(nrows // K1_ROWS,),
        in_specs=[
            pl.BlockSpec(memory_space=pltpu.SMEM),
            pl.BlockSpec((K1_ROWS, ncols), lambda g: (g, 0)),
            pl.BlockSpec((K1_ROWS, ncols), lambda g: (g, 0)),
        ],
        out_specs=pl.BlockSpec((K1_ROWS, ncols), lambda g: (g, 0)),
    )(offsets, xs, ys)
    return lin.reshape(N)


def _sc_body(lin_hbm, feat_hbm, off_hbm, out_hbm,
             lin_buf, filt_ids, id_buf, cell_buf, feat_buf, ones_buf,
             off_buf, s_buf, c_buf, inv_buf, plane_buf, zsum, zcnt,
             sums_sh, cnts_sh, gsem, psem):
    cid = lax.axis_index("c")
    sid = lax.axis_index("s")
    iota = lax.iota(jnp.int32, L)

    def fill_ones(k, _):
        ones_buf[pl.ds(k * L, L)] = jnp.ones((L,), jnp.float32)
        return 0
    lax.fori_loop(0, G // L, fill_ones, 0)

    def fill_zsum(k, _):
        r = k // (DIM // L)
        col = (k % (DIM // L)) * L
        zsum[r, pl.ds(col, L)] = jnp.zeros((L,), jnp.float32)
        return 0
    lax.fori_loop(0, ZR * (DIM // L), fill_zsum, 0)

    def fill_zcnt(k, _):
        zcnt[pl.ds(k * L, L)] = jnp.zeros((L,), jnp.float32)
        return 0
    lax.fori_loop(0, ZR // L, fill_zcnt, 0)

    lin_buf[pl.ds(SENT, L)] = jnp.full((L,), jnp.int32(0x7FFFFFF), jnp.int32)

    pltpu.sync_copy(off_hbm, off_buf.at[pl.ds(0, B)])

    def zero_sub(q, _):
        cell0 = sid * CPT + q * ZR
        pltpu.sync_copy(zsum, sums_sh.at[pl.ds(cell0, ZR)])
        pltpu.sync_copy(zcnt, cnts_sh.at[pl.ds(cell0, ZR)])
        return 0
    lax.fori_loop(0, CPT // ZR, zero_sub, 0)
    plsc.subcore_barrier()

    def do_pass(p, _):
        chunk_id = cid * NPASS + p
        bi = lax.shift_right_logical(chunk_id, 1)

        bvec = jnp.zeros((L,), jnp.int32) + bi
        end_s = plsc.load_gather(off_buf, [bvec])[0]
        sm = plsc.load_gather(off_buf, [jnp.maximum(bvec - 1, 0)])[0]
        start_s = jnp.where(bi == 0, jnp.int32(0), sm)
        start_s = lax.shift_right_logical(start_s, 3) * 8
        total = end_s - start_s
        span = lax.shift_right_logical(total + (NS * L - 1), 8) * L
        nseg = lax.shift_right_logical(span + (SEG - 1), SEGSH)

        def do_seg(h, _):
            ls = start_s + sid * span + h * SEG
            cb = jnp.minimum(lax.shift_right_logical(ls, 3),
                             jnp.int32((N - SEG) // 8)) * 8
            delta = ls - cb
            lim = span - h * SEG
            pltpu.sync_copy(lin_hbm.at[pl.ds(cb, SEG)],
                            lin_buf.at[pl.ds(0, SEG)])

            def filt(i, c):
                q = i * L + iota
                lin = lin_buf[pl.ds(i * L, L)]
                m = ((lax.shift_right_logical(lin, 15) == chunk_id)
                     & (q >= delta) & (q < lim))
                s = plsc.cumsum(m.astype(jnp.int32))
                pos = jnp.where(m, c + s - 1, TRASH)
                plsc.store_scatter(filt_ids, [pos], q)
                return c + s[L - 1]
            c = lax.fori_loop(0, SEG // L, filt, jnp.int32(0))

            def pad(k, _):
                filt_ids[pl.ds(c + k * L, L)] = jnp.full((L,), SENT, jnp.int32)
                return 0
            lax.fori_loop(0, G // L, pad, 0)

            n_g = lax.shift_right_logical(c + (G - 1), 7)

            def group(g, _):
                base = g * G

                def cp(k, _):
                    loc = filt_ids[pl.ds(base + k * L, L)]
                    lin = plsc.load_gather(lin_buf, [loc])
                    m = ((lax.shift_right_logical(lin, 15) == chunk_id)
                         & (loc >= delta) & (loc < lim))
                    cell = jnp.where(m, lin & (CHUNK - 1), jnp.int32(DUMP))
                    gid = jnp.minimum(cb + loc, jnp.int32(N - 1))
                    id_buf[pl.ds(k * L, L)] = gid
                    cell_buf[pl.ds(k * L, L)] = cell
                    return 0
                lax.fori_loop(0, G // L, cp, 0)
                pltpu.async_copy(feat_hbm.at[id_buf], feat_buf, gsem).wait()
                pltpu.sync_copy(feat_buf, sums_sh.at[cell_buf], add=True)
                pltpu.sync_copy(ones_buf, cnts_sh.at[cell_buf], add=True)
                return 0
            lax.fori_loop(0, n_g, group, 0)
            return 0
        lax.fori_loop(0, nseg, do_seg, 0)
        plsc.subcore_barrier()

        b_idx = lax.shift_right_logical(chunk_id, 1)
        yh = chunk_id & 1

        def out_sub(sub, _):
            cell0 = sid * CPT + sub * OSB
            pltpu.sync_copy(sums_sh.at[pl.ds(cell0, OSB)], s_buf)
            pltpu.sync_copy(cnts_sh.at[pl.ds(cell0, OSB)], c_buf)

            def inv_k(k, _):
                cv = c_buf[pl.ds(k * L, L)]
                inv_buf[pl.ds(k * L, L)] = 1.0 / jnp.maximum(cv, 1.0)
                return 0
            lax.fori_loop(0, OSB // L, inv_k, 0)

            out0 = (b_idx * DIM * H * W + yh * CHUNK
                    + sid * CPT + sub * OSB)

            def per_d(d, _):
                def tr(k, _):
                    rows = k * L + iota
                    v = plsc.load_gather(s_buf, [rows, jnp.full((L,), d, jnp.int32)])
                    v = v * inv_buf[pl.ds(k * L, L)]
                    plane_buf[d, pl.ds(k * L, L)] = v
                    return 0
                lax.fori_loop(0, OSB // L, tr, 0)
                off = out0 + d * (H * W)
                pltpu.async_copy(plane_buf.at[d], out_hbm.at[pl.ds(off, OSB)], psem)
                return 0
            lax.fori_loop(0, DIM, per_d, 0)

            def drain(d, _):
                off = out0 + d * (H * W)
                pltpu.make_async_copy(plane_buf.at[d], out_hbm.at[pl.ds(off, OSB)], psem).wait()
                return 0
            lax.fori_loop(0, DIM, drain, 0)

            def rezero(q, _):
                pltpu.sync_copy(zsum, sums_sh.at[pl.ds(cell0 + q * ZR, ZR)])
                pltpu.sync_copy(zcnt, cnts_sh.at[pl.ds(cell0 + q * ZR, ZR)])
                return 0
            lax.fori_loop(0, OSB // ZR, rezero, 0)
            return 0
        lax.fori_loop(0, NOSB, out_sub, 0)
        plsc.subcore_barrier()
        return 0
    lax.fori_loop(0, NPASS, do_pass, 0)


def kernel(events, features, offsets):
    lin = _compute_lin(events, offsets)
    mesh = plsc.VectorSubcoreMesh(core_axis_name="c", subcore_axis_name="s",
                                  num_cores=NC, num_subcores=NS)
    run = pl.kernel(
        _sc_body,
        out_type=jax.ShapeDtypeStruct((B * DIM * H * W,), jnp.float32),
        mesh=mesh,
        scratch_types=[
            pltpu.VMEM((SEG + L,), jnp.int32),
            pltpu.VMEM((FSZ,), jnp.int32),
            pltpu.VMEM((G,), jnp.int32),
            pltpu.VMEM((G,), jnp.int32),
            pltpu.VMEM((G, DIM), jnp.float32),
            pltpu.VMEM((G,), jnp.float32),
            pltpu.VMEM((L,), jnp.int32),
            pltpu.VMEM((OSB, DIM), jnp.float32),
            pltpu.VMEM((OSB,), jnp.float32),
            pltpu.VMEM((OSB,), jnp.float32),
            pltpu.VMEM((DIM, OSB), jnp.float32),
            pltpu.VMEM((ZR, DIM), jnp.float32),
            pltpu.VMEM((ZR,), jnp.float32),
            pltpu.VMEM_SHARED((SROWS, DIM), jnp.float32),
            pltpu.VMEM_SHARED((SROWS,), jnp.float32),
            pltpu.SemaphoreType.DMA,
            pltpu.SemaphoreType.DMA,
        ],
        compiler_params=_SC_PARAMS,
    )
    out = run(lin, features, offsets)
    return out.reshape(B, DIM, H, W)


# speedup vs baseline: 1.8894x; 1.0015x over previous
import jax
import jax.numpy as jnp
from jax import lax
from jax.experimental import pallas as pl
from jax.experimental.pallas import tpu as pltpu
from jax.experimental.pallas import tpu_sc as plsc

H = 256
W = 256
DIM = 32
B = 8
N = 524288

NC = 2
NS = 16
L = 16

CHUNK = 32768
NCHUNK = B * H * W // CHUNK
NPASS = NCHUNK // NC
DUMP = CHUNK
SROWS = CHUNK + 8

SEG = 4096
SEGSH = 12
G = 128
FSZ = SEG + G + 8
TRASH = SEG + G
SENT = SEG

CPT = CHUNK // NS
OSB = 256
NOSB = CPT // OSB
ZR = 128

RNE = 8388608.0

_SC_PARAMS = pltpu.CompilerParams(
    use_tc_tiling_on_sc=False, needs_layout_passes=False)


K1_ROWS = 512


def _lin_body(off_ref, x_ref, y_ref, lin_ref):
    g = pl.program_id(0)
    xv = x_ref[...]
    yv = y_ref[...]
    xr = (xv * float(W) + RNE) - RNE
    yr = (yv * float(H) + RNE) - RNE
    xi = jnp.minimum(jnp.maximum(xr, 0.0), float(W - 1)).astype(jnp.int32)
    yi = jnp.minimum(jnp.maximum(yr, 0.0), float(H - 1)).astype(jnp.int32)
    r = lax.broadcasted_iota(jnp.int32, xv.shape, 0)
    c = lax.broadcasted_iota(jnp.int32, xv.shape, 1)
    j = (g * K1_ROWS + r) * xv.shape[1] + c
    b = jnp.zeros(xv.shape, jnp.int32)
    for k in range(B - 1):
        b = b + jnp.where(j >= off_ref[k], 1, 0).astype(jnp.int32)
    lin_ref[...] = b * (H * W) + yi * W + xi


def _compute_lin(events, offsets):
    ncols = 512
    nrows = N // ncols
    xs = events[:, 0].reshape(nrows, ncols)
    ys = events[:, 1].reshape(nrows, ncols)
    lin = pl.pallas_call(
        _lin_body,
        out_shape=jax.ShapeDtypeStruct((nrows, ncols), jnp.int32),
        grid=(nrows // K1_ROWS,),
        in_specs=[
            pl.BlockSpec(memory_space=pltpu.SMEM),
            pl.BlockSpec((K1_ROWS, ncols), lambda g: (g, 0)),
            pl.BlockSpec((K1_ROWS, ncols), lambda g: (g, 0)),
        ],
        out_specs=pl.BlockSpec((K1_ROWS, ncols), lambda g: (g, 0)),
    )(offsets, xs, ys)
    return lin.reshape(N)


def _sc_body(lin_hbm, feat_hbm, off_hbm, out_hbm,
             lin_buf, filt_ids, id_buf, cell_buf, feat_buf, ones_buf,
             off_buf, s_buf, c_buf, inv_buf, plane_buf, zsum, zcnt,
             sums_sh, cnts_sh, gsem, psem):
    cid = lax.axis_index("c")
    sid = lax.axis_index("s")
    iota = lax.iota(jnp.int32, L)

    def fill_ones(k, _):
        ones_buf[pl.ds(k * L, L)] = jnp.ones((L,), jnp.float32)
        return 0
    lax.fori_loop(0, G // L, fill_ones, 0)

    def fill_zsum(k, _):
        r = k // (DIM // L)
        col = (k % (DIM // L)) * L
        zsum[r, pl.ds(col, L)] = jnp.zeros((L,), jnp.float32)
        return 0
    lax.fori_loop(0, ZR * (DIM // L), fill_zsum, 0)

    def fill_zcnt(k, _):
        zcnt[pl.ds(k * L, L)] = jnp.zeros((L,), jnp.float32)
        return 0
    lax.fori_loop(0, ZR // L, fill_zcnt, 0)

    lin_buf[pl.ds(SENT, L)] = jnp.full((L,), jnp.int32(0x7FFFFFF), jnp.int32)

    pltpu.sync_copy(off_hbm, off_buf.at[pl.ds(0, B)])

    def zero_sub(q, _):
        cell0 = sid * CPT + q * ZR
        pltpu.sync_copy(zsum, sums_sh.at[pl.ds(cell0, ZR)])
        pltpu.sync_copy(zcnt, cnts_sh.at[pl.ds(cell0, ZR)])
        return 0
    lax.fori_loop(0, CPT // ZR, zero_sub, 0)
    plsc.subcore_barrier()

    def do_pass(p, _):
        chunk_id = cid * NPASS + p
        bi = lax.shift_right_logical(chunk_id, 1)

        bvec = jnp.zeros((L,), jnp.int32) + bi
        end_s = plsc.load_gather(off_buf, [bvec])[0]
        sm = plsc.load_gather(off_buf, [jnp.maximum(bvec - 1, 0)])[0]
        start_s = jnp.where(bi == 0, jnp.int32(0), sm)
        start_s = lax.shift_right_logical(start_s, 3) * 8
        total = end_s - start_s
        span = lax.shift_right_logical(total + (NS * L - 1), 8) * L
        nseg = lax.shift_right_logical(span + (SEG - 1), SEGSH)

        def do_seg(h, _):
            ls = start_s + sid * span + h * SEG
            cb = jnp.minimum(lax.shift_right_logical(ls, 3),
                             jnp.int32((N - SEG) // 8)) * 8
            delta = ls - cb
            lim = span - h * SEG + delta
            pltpu.sync_copy(lin_hbm.at[pl.ds(cb, SEG)],
                            lin_buf.at[pl.ds(0, SEG)])

            def filt(i, c):
                q = i * L + iota
                lin = lin_buf[pl.ds(i * L, L)]
                m = ((lax.shift_right_logical(lin, 15) == chunk_id)
                     & (q >= delta) & (q < lim))
                s = plsc.cumsum(m.astype(jnp.int32))
                pos = jnp.where(m, c + s - 1, TRASH)
                plsc.store_scatter(filt_ids, [pos], q)
                return c + s[L - 1]
            c = lax.fori_loop(0, SEG // L, filt, jnp.int32(0))

            def pad(k, _):
                filt_ids[pl.ds(c + k * L, L)] = jnp.full((L,), SENT, jnp.int32)
                return 0
            lax.fori_loop(0, G // L, pad, 0)

            n_g = lax.shift_right_logical(c + (G - 1), 7)

            def group(g, _):
                base = g * G

                def cp(k, _):
                    loc = filt_ids[pl.ds(base + k * L, L)]
                    lin = plsc.load_gather(lin_buf, [loc])
                    m = ((lax.shift_right_logical(lin, 15) == chunk_id)
                         & (loc >= delta) & (loc < lim))
                    cell = jnp.where(m, lin & (CHUNK - 1), jnp.int32(DUMP))
                    gid = jnp.minimum(cb + loc, jnp.int32(N - 1))
                    id_buf[pl.ds(k * L, L)] = gid
                    cell_buf[pl.ds(k * L, L)] = cell
                    return 0
                lax.fori_loop(0, G // L, cp, 0)
                pltpu.async_copy(feat_hbm.at[id_buf], feat_buf, gsem).wait()
                pltpu.sync_copy(feat_buf, sums_sh.at[cell_buf], add=True)
                pltpu.sync_copy(ones_buf, cnts_sh.at[cell_buf], add=True)
                return 0
            lax.fori_loop(0, n_g, group, 0)
            return 0
        lax.fori_loop(0, nseg, do_seg, 0)
        plsc.subcore_barrier()

        b_idx = lax.shift_right_logical(chunk_id, 1)
        yh = chunk_id & 1

        def out_sub(sub, _):
            cell0 = sid * CPT + sub * OSB
            pltpu.sync_copy(sums_sh.at[pl.ds(cell0, OSB)], s_buf)
            pltpu.sync_copy(cnts_sh.at[pl.ds(cell0, OSB)], c_buf)

            def inv_k(k, _):
                cv = c_buf[pl.ds(k * L, L)]
                inv_buf[pl.ds(k * L, L)] = 1.0 / jnp.maximum(cv, 1.0)
                return 0
            lax.fori_loop(0, OSB // L, inv_k, 0)

            out0 = (b_idx * DIM * H * W + yh * CHUNK
                    + sid * CPT + sub * OSB)

            def per_d(d, _):
                def tr(k, _):
                    rows = k * L + iota
                    v = plsc.load_gather(s_buf, [rows, jnp.full((L,), d, jnp.int32)])
                    v = v * inv_buf[pl.ds(k * L, L)]
                    plane_buf[d, pl.ds(k * L, L)] = v
                    return 0
                lax.fori_loop(0, OSB // L, tr, 0)
                off = out0 + d * (H * W)
                pltpu.async_copy(plane_buf.at[d], out_hbm.at[pl.ds(off, OSB)], psem)
                return 0
            lax.fori_loop(0, DIM, per_d, 0)

            def drain(d, _):
                off = out0 + d * (H * W)
                pltpu.make_async_copy(plane_buf.at[d], out_hbm.at[pl.ds(off, OSB)], psem).wait()
                return 0
            lax.fori_loop(0, DIM, drain, 0)

            def rezero(q, _):
                pltpu.sync_copy(zsum, sums_sh.at[pl.ds(cell0 + q * ZR, ZR)])
                pltpu.sync_copy(zcnt, cnts_sh.at[pl.ds(cell0 + q * ZR, ZR)])
                return 0
            lax.fori_loop(0, OSB // ZR, rezero, 0)
            return 0
        lax.fori_loop(0, NOSB, out_sub, 0)
        plsc.subcore_barrier()
        return 0
    lax.fori_loop(0, NPASS, do_pass, 0)


def kernel(events, features, offsets):
    lin = _compute_lin(events, offsets)
    mesh = plsc.VectorSubcoreMesh(core_axis_name="c", subcore_axis_name="s",
                                  num_cores=NC, num_subcores=NS)
    run = pl.kernel(
        _sc_body,
        out_type=jax.ShapeDtypeStruct((B * DIM * H * W,), jnp.float32),
        mesh=mesh,
        scratch_types=[
            pltpu.VMEM((SEG + L,), jnp.int32),
            pltpu.VMEM((FSZ,), jnp.int32),
            pltpu.VMEM((G,), jnp.int32),
            pltpu.VMEM((G,), jnp.int32),
            pltpu.VMEM((G, DIM), jnp.float32),
            pltpu.VMEM((G,), jnp.float32),
            pltpu.VMEM((L,), jnp.int32),
            pltpu.VMEM((OSB, DIM), jnp.float32),
            pltpu.VMEM((OSB,), jnp.float32),
            pltpu.VMEM((OSB,), jnp.float32),
            pltpu.VMEM((DIM, OSB), jnp.float32),
            pltpu.VMEM((ZR, DIM), jnp.float32),
            pltpu.VMEM((ZR,), jnp.float32),
            pltpu.VMEM_SHARED((SROWS, DIM), jnp.float32),
            pltpu.VMEM_SHARED((SROWS,), jnp.float32),
            pltpu.SemaphoreType.DMA,
            pltpu.SemaphoreType.DMA,
        ],
        compiler_params=_SC_PARAMS,
    )
    out = run(lin, features, offsets)
    return out.reshape(B, DIM, H, W)


# speedup vs baseline: 1.9024x; 1.0069x over previous
import jax
import jax.numpy as jnp
from jax import lax
from jax.experimental import pallas as pl
from jax.experimental.pallas import tpu as pltpu
from jax.experimental.pallas import tpu_sc as plsc

H = 256
W = 256
DIM = 32
B = 8
N = 524288

NC = 2
NS = 16
L = 16

CHUNK = 32768
NCHUNK = B * H * W // CHUNK
NPASS = NCHUNK // NC
DUMP = CHUNK
SROWS = CHUNK + 8

SEG = 4096
SEGSH = 12
G = 256
GSH = 8
FSZ = SEG + G + 8
TRASH = SEG + G
SENT = SEG

CPT = CHUNK // NS
OSB = 256
NOSB = CPT // OSB
ZR = 128

RNE = 8388608.0

_SC_PARAMS = pltpu.CompilerParams(
    use_tc_tiling_on_sc=False, needs_layout_passes=False)


K1_ROWS = 512


def _lin_body(off_ref, x_ref, y_ref, lin_ref):
    g = pl.program_id(0)
    xv = x_ref[...]
    yv = y_ref[...]
    xr = (xv * float(W) + RNE) - RNE
    yr = (yv * float(H) + RNE) - RNE
    xi = jnp.minimum(jnp.maximum(xr, 0.0), float(W - 1)).astype(jnp.int32)
    yi = jnp.minimum(jnp.maximum(yr, 0.0), float(H - 1)).astype(jnp.int32)
    r = lax.broadcasted_iota(jnp.int32, xv.shape, 0)
    c = lax.broadcasted_iota(jnp.int32, xv.shape, 1)
    j = (g * K1_ROWS + r) * xv.shape[1] + c
    b = jnp.zeros(xv.shape, jnp.int32)
    for k in range(B - 1):
        b = b + jnp.where(j >= off_ref[k], 1, 0).astype(jnp.int32)
    lin_ref[...] = b * (H * W) + yi * W + xi


def _compute_lin(events, offsets):
    ncols = 512
    nrows = N // ncols
    xs = events[:, 0].reshape(nrows, ncols)
    ys = events[:, 1].reshape(nrows, ncols)
    lin = pl.pallas_call(
        _lin_body,
        out_shape=jax.ShapeDtypeStruct((nrows, ncols), jnp.int32),
        grid=(nrows // K1_ROWS,),
        in_specs=[
            pl.BlockSpec(memory_space=pltpu.SMEM),
            pl.BlockSpec((K1_ROWS, ncols), lambda g: (g, 0)),
            pl.BlockSpec((K1_ROWS, ncols), lambda g: (g, 0)),
        ],
        out_specs=pl.BlockSpec((K1_ROWS, ncols), lambda g: (g, 0)),
    )(offsets, xs, ys)
    return lin.reshape(N)


def _sc_body(lin_hbm, feat_hbm, off_hbm, out_hbm,
             lin_buf, filt_ids, id_buf, cell_buf, feat_buf, ones_buf,
             off_buf, s_buf, c_buf, inv_buf, plane_buf, zsum, zcnt,
             sums_sh, cnts_sh, gsem, psem):
    cid = lax.axis_index("c")
    sid = lax.axis_index("s")
    iota = lax.iota(jnp.int32, L)

    def fill_ones(k, _):
        ones_buf[pl.ds(k * L, L)] = jnp.ones((L,), jnp.float32)
        return 0
    lax.fori_loop(0, G // L, fill_ones, 0)

    def fill_zsum(k, _):
        r = k // (DIM // L)
        col = (k % (DIM // L)) * L
        zsum[r, pl.ds(col, L)] = jnp.zeros((L,), jnp.float32)
        return 0
    lax.fori_loop(0, ZR * (DIM // L), fill_zsum, 0)

    def fill_zcnt(k, _):
        zcnt[pl.ds(k * L, L)] = jnp.zeros((L,), jnp.float32)
        return 0
    lax.fori_loop(0, ZR // L, fill_zcnt, 0)

    lin_buf[pl.ds(SENT, L)] = jnp.full((L,), jnp.int32(0x7FFFFFF), jnp.int32)

    pltpu.sync_copy(off_hbm, off_buf.at[pl.ds(0, B)])

    def zero_sub(q, _):
        cell0 = sid * CPT + q * ZR
        pltpu.sync_copy(zsum, sums_sh.at[pl.ds(cell0, ZR)])
        pltpu.sync_copy(zcnt, cnts_sh.at[pl.ds(cell0, ZR)])
        return 0
    lax.fori_loop(0, CPT // ZR, zero_sub, 0)
    plsc.subcore_barrier()

    def do_pass(p, _):
        chunk_id = cid * NPASS + p
        bi = lax.shift_right_logical(chunk_id, 1)

        bvec = jnp.zeros((L,), jnp.int32) + bi
        end_s = plsc.load_gather(off_buf, [bvec])[0]
        sm = plsc.load_gather(off_buf, [jnp.maximum(bvec - 1, 0)])[0]
        start_s = jnp.where(bi == 0, jnp.int32(0), sm)
        start_s = lax.shift_right_logical(start_s, 3) * 8
        total = end_s - start_s
        span = lax.shift_right_logical(total + (NS * L - 1), 8) * L
        nseg = lax.shift_right_logical(span + (SEG - 1), SEGSH)

        def do_seg(h, _):
            ls = start_s + sid * span + h * SEG
            cb = jnp.minimum(lax.shift_right_logical(ls, 3),
                             jnp.int32((N - SEG) // 8)) * 8
            delta = ls - cb
            lim = span - h * SEG + delta
            pltpu.sync_copy(lin_hbm.at[pl.ds(cb, SEG)],
                            lin_buf.at[pl.ds(0, SEG)])

            def filt(i, c):
                q = i * L + iota
                lin = lin_buf[pl.ds(i * L, L)]
                m = ((lax.shift_right_logical(lin, 15) == chunk_id)
                     & (q >= delta) & (q < lim))
                s = plsc.cumsum(m.astype(jnp.int32))
                pos = jnp.where(m, c + s - 1, TRASH)
                plsc.store_scatter(filt_ids, [pos], q)
                return c + s[L - 1]
            c = lax.fori_loop(0, SEG // L, filt, jnp.int32(0))

            def pad(k, _):
                filt_ids[pl.ds(c + k * L, L)] = jnp.full((L,), SENT, jnp.int32)
                return 0
            lax.fori_loop(0, G // L, pad, 0)

            n_g = lax.shift_right_logical(c + (G - 1), GSH)

            def group(g, _):
                base = g * G

                def cp(k, _):
                    loc = filt_ids[pl.ds(base + k * L, L)]
                    lin = plsc.load_gather(lin_buf, [loc])
                    m = ((lax.shift_right_logical(lin, 15) == chunk_id)
                         & (loc >= delta) & (loc < lim))
                    cell = jnp.where(m, lin & (CHUNK - 1), jnp.int32(DUMP))
                    gid = jnp.minimum(cb + loc, jnp.int32(N - 1))
                    id_buf[pl.ds(k * L, L)] = gid
                    cell_buf[pl.ds(k * L, L)] = cell
                    return 0
                lax.fori_loop(0, G // L, cp, 0)
                pltpu.async_copy(feat_hbm.at[id_buf], feat_buf, gsem).wait()
                pltpu.sync_copy(feat_buf, sums_sh.at[cell_buf], add=True)
                pltpu.sync_copy(ones_buf, cnts_sh.at[cell_buf], add=True)
                return 0
            lax.fori_loop(0, n_g, group, 0)
            return 0
        lax.fori_loop(0, nseg, do_seg, 0)
        plsc.subcore_barrier()

        b_idx = lax.shift_right_logical(chunk_id, 1)
        yh = chunk_id & 1

        def out_sub(sub, _):
            cell0 = sid * CPT + sub * OSB
            pltpu.sync_copy(sums_sh.at[pl.ds(cell0, OSB)], s_buf)
            pltpu.sync_copy(cnts_sh.at[pl.ds(cell0, OSB)], c_buf)

            def inv_k(k, _):
                cv = c_buf[pl.ds(k * L, L)]
                inv_buf[pl.ds(k * L, L)] = 1.0 / jnp.maximum(cv, 1.0)
                return 0
            lax.fori_loop(0, OSB // L, inv_k, 0)

            out0 = (b_idx * DIM * H * W + yh * CHUNK
                    + sid * CPT + sub * OSB)

            def per_d(d, _):
                def tr(k, _):
                    rows = k * L + iota
                    v = plsc.load_gather(s_buf, [rows, jnp.full((L,), d, jnp.int32)])
                    v = v * inv_buf[pl.ds(k * L, L)]
                    plane_buf[d, pl.ds(k * L, L)] = v
                    return 0
                lax.fori_loop(0, OSB // L, tr, 0)
                off = out0 + d * (H * W)
                pltpu.async_copy(plane_buf.at[d], out_hbm.at[pl.ds(off, OSB)], psem)
                return 0
            lax.fori_loop(0, DIM, per_d, 0)

            def drain(d, _):
                off = out0 + d * (H * W)
                pltpu.make_async_copy(plane_buf.at[d], out_hbm.at[pl.ds(off, OSB)], psem).wait()
                return 0
            lax.fori_loop(0, DIM, drain, 0)

            def rezero(q, _):
                pltpu.sync_copy(zsum, sums_sh.at[pl.ds(cell0 + q * ZR, ZR)])
                pltpu.sync_copy(zcnt, cnts_sh.at[pl.ds(cell0 + q * ZR, ZR)])
                return 0
            lax.fori_loop(0, OSB // ZR, rezero, 0)
            return 0
        lax.fori_loop(0, NOSB, out_sub, 0)
        plsc.subcore_barrier()
        return 0
    lax.fori_loop(0, NPASS, do_pass, 0)


def kernel(events, features, offsets):
    lin = _compute_lin(events, offsets)
    mesh = plsc.VectorSubcoreMesh(core_axis_name="c", subcore_axis_name="s",
                                  num_cores=NC, num_subcores=NS)
    run = pl.kernel(
        _sc_body,
        out_type=jax.ShapeDtypeStruct((B * DIM * H * W,), jnp.float32),
        mesh=mesh,
        scratch_types=[
            pltpu.VMEM((SEG + L,), jnp.int32),
            pltpu.VMEM((FSZ,), jnp.int32),
            pltpu.VMEM((G,), jnp.int32),
            pltpu.VMEM((G,), jnp.int32),
            pltpu.VMEM((G, DIM), jnp.float32),
            pltpu.VMEM((G,), jnp.float32),
            pltpu.VMEM((L,), jnp.int32),
            pltpu.VMEM((OSB, DIM), jnp.float32),
            pltpu.VMEM((OSB,), jnp.float32),
            pltpu.VMEM((OSB,), jnp.float32),
            pltpu.VMEM((DIM, OSB), jnp.float32),
            pltpu.VMEM((ZR, DIM), jnp.float32),
            pltpu.VMEM((ZR,), jnp.float32),
            pltpu.VMEM_SHARED((SROWS, DIM), jnp.float32),
            pltpu.VMEM_SHARED((SROWS,), jnp.float32),
            pltpu.SemaphoreType.DMA,
            pltpu.SemaphoreType.DMA,
        ],
        compiler_params=_SC_PARAMS,
    )
    out = run(lin, features, offsets)
    return out.reshape(B, DIM, H, W)
